# baseline (speedup 1.0000x reference)
import jax
import jax.numpy as jnp
from jax.experimental import pallas as pl
from jax.experimental.pallas import tpu as pltpu

_K = 300
_KPAD = 384
_C = 91
_ROWS = 14336
_G = _ROWS // 8
_NBI = 4
_PAD_VAL = -1.0
_DONE_VAL = -2.0


def _select_body(prob_hbm, scores_ref, labels_ref, qidx_ref, prob_ref, sem):
    gb = pl.program_id(0)
    copy = pltpu.make_async_copy(
        prob_hbm.at[pl.ds(gb * _NBI, _NBI)], prob_ref, sem
    )
    copy.start()
    copy.wait()
    scores_ref[...] = jnp.zeros_like(scores_ref)
    labels_ref[...] = jnp.zeros_like(labels_ref)
    qidx_ref[...] = jnp.zeros_like(qidx_ref)

    sr_iota = (
        jax.lax.broadcasted_iota(jnp.int32, (14, 128), 0) * 128
        + jax.lax.broadcasted_iota(jnp.int32, (14, 128), 1)
    )
    blk_iota = (
        jax.lax.broadcasted_iota(jnp.int32, (8, 128), 0) * 128
        + jax.lax.broadcasted_iota(jnp.int32, (8, 128), 1)
    )
    big = jnp.int32(2**30)

    sr0 = tuple(
        prob_ref[i].reshape(_G, 8, 128).max(axis=(1, 2)).reshape(14, 128)
        for i in range(_NBI)
    )

    def one(i, j, sr):
        m = jnp.max(sr)
        g = jnp.min(jnp.where(sr == m, sr_iota, big))
        blk = prob_ref[i, pl.ds(g * 8, 8), :]
        v = jnp.max(blk)
        loc = jnp.min(jnp.where(blk == v, blk_iota, big))
        gi = g * 1024 + loc
        blk2 = jnp.where(blk_iota == loc, jnp.float32(_DONE_VAL), blk)
        prob_ref[i, pl.ds(g * 8, 8), :] = blk2
        sr = jnp.where(sr_iota == g, jnp.max(blk2), sr)
        scores_ref[i, pl.ds(j, 1), :] = v.reshape(1, 1)
        labels_ref[i, pl.ds(j, 1), :] = (gi % _C).reshape(1, 1)
        qidx_ref[i, pl.ds(j, 1), :] = (gi // _C).reshape(1, 1)
        return sr

    def step(j, srs):
        return tuple(one(i, j, srs[i]) for i in range(_NBI))

    jax.lax.fori_loop(0, _K, step, sr0)


def _gather_body(ts_ref, qidx_ref, boxes_ref, bx_out_ref):
    b = pl.program_id(0)
    bx_out_ref[...] = jnp.zeros_like(bx_out_ref)
    img_h = ts_ref[b, 0].astype(jnp.float32)
    img_w = ts_ref[b, 1].astype(jnp.float32)
    s2 = jnp.concatenate(
        [jnp.full((1, 1), img_w, jnp.float32), jnp.full((1, 1), img_h, jnp.float32)],
        axis=1,
    )

    def step(j, carry):
        q = qidx_ref[0, pl.ds(j, 1), :][0, 0]
        bx = boxes_ref[0, pl.ds(q, 1), :]
        a = bx[:, 0:2]
        d = bx[:, 2:4]
        bx_out_ref[0, pl.ds(j, 1), 0:2] = (a - 0.5 * d) * s2
        bx_out_ref[0, pl.ds(j, 1), 2:4] = (a + 0.5 * d) * s2
        return carry

    jax.lax.fori_loop(0, _K, step, 0)


def kernel(pred_logits, pred_boxes, target_sizes):
    B, Q, C = pred_logits.shape
    n = Q * C
    prob = jax.nn.sigmoid(pred_logits).reshape(B, n)
    probp = jnp.pad(prob, ((0, 0), (0, _ROWS * 128 - n)), constant_values=_PAD_VAL)
    prob3 = probp.reshape(B, _ROWS, 128)

    scores, labels, qidx = pl.pallas_call(
        _select_body,
        grid=(B // _NBI,),
        in_specs=[
            pl.BlockSpec(memory_space=pl.ANY),
        ],
        scratch_shapes=[
            pltpu.VMEM((_NBI, _ROWS, 128), jnp.float32),
            pltpu.SemaphoreType.DMA,
        ],
        out_specs=[
            pl.BlockSpec((_NBI, _KPAD, 1), lambda b: (b, 0, 0)),
            pl.BlockSpec((_NBI, _KPAD, 1), lambda b: (b, 0, 0)),
            pl.BlockSpec((_NBI, _KPAD, 1), lambda b: (b, 0, 0)),
        ],
        out_shape=[
            jax.ShapeDtypeStruct((B, _KPAD, 1), jnp.float32),
            jax.ShapeDtypeStruct((B, _KPAD, 1), jnp.int32),
            jax.ShapeDtypeStruct((B, _KPAD, 1), jnp.int32),
        ],
    )(prob3)

    boxes = pl.pallas_call(
        _gather_body,
        grid=(B,),
        in_specs=[
            pl.BlockSpec(memory_space=pltpu.SMEM),
            pl.BlockSpec((1, _KPAD, 1), lambda b: (b, 0, 0)),
            pl.BlockSpec((1, Q, 4), lambda b: (b, 0, 0)),
        ],
        out_specs=pl.BlockSpec((1, _KPAD, 4), lambda b: (b, 0, 0)),
        out_shape=jax.ShapeDtypeStruct((B, _KPAD, 4), jnp.float32),
    )(target_sizes, qidx, pred_boxes)

    return (scores[:, :_K, 0], labels[:, :_K, 0], boxes[:, :_K, :])


# speedup vs baseline: 1.1236x; 1.1236x over previous
import jax
import jax.numpy as jnp
from jax.experimental import pallas as pl
from jax.experimental.pallas import tpu as pltpu

_K = 300
_KPAD = 384
_C = 91
_ROWS = 14336
_G = _ROWS // 8
_PAD_VAL = -1.0
_DONE_VAL = -2.0


def _body(ts_ref, prob_ref, boxes_ref, scores_ref, labels_ref, bx_out_ref):
    b = pl.program_id(0)
    scores_ref[...] = jnp.zeros_like(scores_ref)
    labels_ref[...] = jnp.zeros_like(labels_ref)
    bx_out_ref[...] = jnp.zeros_like(bx_out_ref)

    img_h = ts_ref[b, 0].astype(jnp.float32)
    img_w = ts_ref[b, 1].astype(jnp.float32)
    s2 = jnp.concatenate(
        [jnp.full((1, 1), img_w, jnp.float32), jnp.full((1, 1), img_h, jnp.float32)],
        axis=1,
    )

    x = prob_ref[0]
    sr = x.reshape(_G, 8, 128).max(axis=(1, 2)).reshape(14, 128)

    sr_iota = (
        jax.lax.broadcasted_iota(jnp.int32, (14, 128), 0) * 128
        + jax.lax.broadcasted_iota(jnp.int32, (14, 128), 1)
    )
    blk_iota = (
        jax.lax.broadcasted_iota(jnp.int32, (8, 128), 0) * 128
        + jax.lax.broadcasted_iota(jnp.int32, (8, 128), 1)
    )

    big = jnp.int32(2**30)

    def step(j, sr):
        m = jnp.max(sr)
        g = jnp.min(jnp.where(sr == m, sr_iota, big))
        blk = prob_ref[0, pl.ds(g * 8, 8), :]
        v = m
        loc = jnp.min(jnp.where(blk == v, blk_iota, big))
        gi = g * 1024 + loc
        q = gi // _C
        c = gi % _C

        blk2 = jnp.where(blk_iota == loc, jnp.float32(_DONE_VAL), blk)
        prob_ref[0, pl.ds(g * 8, 8), :] = blk2
        sr = jnp.where(sr_iota == g, jnp.max(blk2), sr)

        scores_ref[0, pl.ds(j, 1), :] = v.reshape(1, 1)
        labels_ref[0, pl.ds(j, 1), :] = c.reshape(1, 1)

        bx = boxes_ref[0, pl.ds(q, 1), :]
        a = bx[:, 0:2]
        d = bx[:, 2:4]
        bx_out_ref[0, pl.ds(j, 1), 0:2] = (a - 0.5 * d) * s2
        bx_out_ref[0, pl.ds(j, 1), 2:4] = (a + 0.5 * d) * s2
        return sr

    jax.lax.fori_loop(0, _K, step, sr)


def kernel(pred_logits, pred_boxes, target_sizes):
    B, Q, C = pred_logits.shape
    n = Q * C
    prob = jax.nn.sigmoid(pred_logits).reshape(B, n)
    probp = jnp.pad(prob, ((0, 0), (0, _ROWS * 128 - n)), constant_values=_PAD_VAL)
    prob3 = probp.reshape(B, _ROWS, 128)

    scores, labels, boxes = pl.pallas_call(
        _body,
        grid=(B,),
        in_specs=[
            pl.BlockSpec(memory_space=pltpu.SMEM),
            pl.BlockSpec((1, _ROWS, 128), lambda b: (b, 0, 0)),
            pl.BlockSpec((1, Q, 4), lambda b: (b, 0, 0)),
        ],
        out_specs=[
            pl.BlockSpec((1, _KPAD, 1), lambda b: (b, 0, 0)),
            pl.BlockSpec((1, _KPAD, 1), lambda b: (b, 0, 0)),
            pl.BlockSpec((1, _KPAD, 4), lambda b: (b, 0, 0)),
        ],
        out_shape=[
            jax.ShapeDtypeStruct((B, _KPAD, 1), jnp.float32),
            jax.ShapeDtypeStruct((B, _KPAD, 1), jnp.int32),
            jax.ShapeDtypeStruct((B, _KPAD, 4), jnp.float32),
        ],
    )(target_sizes, prob3, pred_boxes)

    return (scores[:, :_K, 0], labels[:, :_K, 0], boxes[:, :_K, :])
